# baseline (speedup 1.0000x reference)
import jax
import jax.numpy as jnp
from jax import lax
from jax.experimental import pallas as pl
from jax.experimental.pallas import tpu as pltpu
from jax.experimental.pallas import tpu_sc as plsc

B, L, D = 16, 4096, 128
NUM_SEGMENTS = 1024

NUM_CORES = 2
NUM_SUBCORES = 16
LANES = 16

BATCH_PER_CORE = B // NUM_CORES
TILES_PER_BATCH = NUM_SUBCORES // BATCH_PER_CORE
ROWS_PER_TILE = L // TILES_PER_BATCH
CHUNK = 128
NCHUNK = ROWS_PER_TILE // CHUNK
SEG_PER_TILE = NUM_SEGMENTS // TILES_PER_BATCH
ZROWS = 32


def _sc_body(v_hbm, seg_hbm, out_hbm, vbuf0, vbuf1, idx_all, zbuf,
             lsem0, lsem1, ssem, isem, zsem, acc):
    c = lax.axis_index("c")
    s = lax.axis_index("s")
    b_local = s // TILES_PER_BATCH
    half = s % TILES_PER_BATCH
    b = c * BATCH_PER_CORE + b_local
    row0 = half * ROWS_PER_TILE
    tbl_base = b_local * NUM_SEGMENTS
    acc_base = tbl_base + half * SEG_PER_TILE

    def start_load(k, vbuf, lsem):
        pltpu.async_copy(v_hbm.at[b, pl.ds(row0 + k * CHUNK, CHUNK)],
                         vbuf, lsem)

    def wait_load(vbuf, lsem):
        pltpu.make_async_copy(v_hbm.at[b, pl.ds(row0, CHUNK)],
                              vbuf, lsem).wait()

    start_load(0, vbuf0, lsem0)
    start_load(1, vbuf1, lsem1)

    @pl.loop(0, NCHUNK)
    def _(k):
        pltpu.async_copy(
            seg_hbm.at[b, pl.ds(row0 + k * CHUNK, CHUNK)],
            idx_all.at[k], isem)

    @pl.loop(0, ZROWS)
    def _(i):
        for j in range(D // LANES):
            zbuf[i, pl.ds(j * LANES, LANES)] = jnp.zeros((LANES,),
                                                         jnp.float32)

    @pl.loop(0, SEG_PER_TILE // ZROWS)
    def _(r):
        pltpu.async_copy(
            zbuf, acc.at[pl.ds(acc_base + r * ZROWS, ZROWS)], zsem)

    @pl.loop(0, NCHUNK)
    def _(k):
        pltpu.make_async_copy(
            seg_hbm.at[b, pl.ds(row0, CHUNK)], idx_all.at[0], isem).wait()

    @pl.loop(0, NCHUNK)
    def _(k):
        for j in range(CHUNK // LANES):
            sl = pl.ds(j * LANES, LANES)
            idx_all[k, sl] = idx_all[k, sl] + tbl_base

    @pl.loop(0, SEG_PER_TILE // ZROWS)
    def _(r):
        pltpu.make_async_copy(
            zbuf, acc.at[pl.ds(acc_base, ZROWS)], zsem).wait()
    plsc.subcore_barrier()

    def scat(k, vbuf):
        pltpu.async_copy(vbuf, acc.at[idx_all.at[k]], ssem, add=True).wait()

    @pl.loop(0, NCHUNK // 2 - 1)
    def _(g):
        k0 = 2 * g
        wait_load(vbuf0, lsem0)
        scat(k0, vbuf0)
        start_load(k0 + 2, vbuf0, lsem0)
        wait_load(vbuf1, lsem1)
        scat(k0 + 1, vbuf1)
        start_load(k0 + 3, vbuf1, lsem1)

    wait_load(vbuf0, lsem0)
    scat(NCHUNK - 2, vbuf0)
    wait_load(vbuf1, lsem1)
    scat(NCHUNK - 1, vbuf1)

    plsc.subcore_barrier()
    pltpu.sync_copy(
        acc.at[pl.ds(acc_base, SEG_PER_TILE)],
        out_hbm.at[b, pl.ds(half * SEG_PER_TILE, SEG_PER_TILE)])


def kernel(data, v, segment_index):
    assert data.shape == v.shape
    seg32 = segment_index.astype(jnp.int32)
    mesh = plsc.VectorSubcoreMesh(
        core_axis_name="c", subcore_axis_name="s",
        num_cores=NUM_CORES, num_subcores=NUM_SUBCORES,
    )
    out = pl.kernel(
        _sc_body,
        out_type=jax.ShapeDtypeStruct((B, NUM_SEGMENTS, D), jnp.float32),
        mesh=mesh,
        scratch_types=[
            pltpu.VMEM((CHUNK, D), jnp.float32),
            pltpu.VMEM((CHUNK, D), jnp.float32),
            pltpu.VMEM((NCHUNK, CHUNK), jnp.int32),
            pltpu.VMEM((ZROWS, D), jnp.float32),
            pltpu.SemaphoreType.DMA,
            pltpu.SemaphoreType.DMA,
            pltpu.SemaphoreType.DMA,
            pltpu.SemaphoreType.DMA,
            pltpu.SemaphoreType.DMA,
            pltpu.VMEM_SHARED((BATCH_PER_CORE * NUM_SEGMENTS, D), jnp.float32),
        ],
    )(v, seg32)
    return out


# speedup vs baseline: 1.0597x; 1.0597x over previous
import jax
import jax.numpy as jnp
from jax import lax
from jax.experimental import pallas as pl
from jax.experimental.pallas import tpu as pltpu
from jax.experimental.pallas import tpu_sc as plsc

B, L, D = 16, 4096, 128
NUM_SEGMENTS = 1024

NUM_CORES = 2
NUM_SUBCORES = 16
LANES = 16

BATCH_PER_CORE = B // NUM_CORES
TILES_PER_BATCH = NUM_SUBCORES // BATCH_PER_CORE
ROWS_PER_TILE = L // TILES_PER_BATCH
CHUNK = 128
NCHUNK = ROWS_PER_TILE // CHUNK
SEG_PER_TILE = NUM_SEGMENTS // TILES_PER_BATCH
NBUF = 3
SLACK = 1
ZROWS = 32


def _sc_body(v_hbm, seg_hbm, out_hbm, vbufs, idx_all, zbuf,
             lsems, ssems, zsem, acc):
    c = lax.axis_index("c")
    s = lax.axis_index("s")
    b_local = s // TILES_PER_BATCH
    half = s % TILES_PER_BATCH
    b = c * BATCH_PER_CORE + b_local
    row0 = half * ROWS_PER_TILE
    tbl_base = b_local * NUM_SEGMENTS
    acc_base = tbl_base + half * SEG_PER_TILE

    ldescs = {}

    def start_load(k):
        off = row0 + k * CHUNK
        ldescs[k] = pltpu.async_copy(
            v_hbm.at[b, pl.ds(off, CHUNK)], vbufs[k % NBUF], lsems[k % NBUF])

    for j in range(NBUF):
        start_load(j)
    idx_descs = [
        pltpu.async_copy(
            seg_hbm.at[b, pl.ds(half * NCHUNK, NCHUNK)], idx_all, ssems[1])
    ]

    @pl.loop(0, ZROWS)
    def _(i):
        for j in range(D // LANES):
            zbuf[i, pl.ds(j * LANES, LANES)] = jnp.zeros((LANES,), jnp.float32)
    zdescs = [
        pltpu.async_copy(zbuf, acc.at[pl.ds(acc_base + r * ZROWS, ZROWS)], zsem)
        for r in range(SEG_PER_TILE // ZROWS)
    ]

    for d in idx_descs:
        d.wait()
    @pl.loop(0, NCHUNK)
    def _(k):
        for j in range(CHUNK // LANES):
            sl = pl.ds(j * LANES, LANES)
            idx_all[k, sl] = idx_all[k, sl] + tbl_base

    for d in zdescs:
        d.wait()
    plsc.subcore_barrier()

    for k in range(NCHUNK):
        ldescs[k].wait()
        pltpu.async_copy(
            vbufs[k % NBUF], acc.at[idx_all.at[k]], ssems[0], add=True).wait()
        if k + NBUF < NCHUNK:
            start_load(k + NBUF)

    plsc.subcore_barrier()
    pltpu.sync_copy(
        acc.at[pl.ds(acc_base, SEG_PER_TILE)],
        out_hbm.at[b, pl.ds(half * SEG_PER_TILE, SEG_PER_TILE)])


def kernel(data, v, segment_index):
    assert data.shape == v.shape
    seg32 = segment_index.astype(jnp.int32).reshape(
        B, TILES_PER_BATCH * NCHUNK, CHUNK)
    mesh = plsc.VectorSubcoreMesh(
        core_axis_name="c", subcore_axis_name="s",
        num_cores=NUM_CORES, num_subcores=NUM_SUBCORES,
    )
    out = pl.kernel(
        _sc_body,
        out_type=jax.ShapeDtypeStruct((B, NUM_SEGMENTS, D), jnp.float32),
        mesh=mesh,
        scratch_types=[
            [pltpu.VMEM((CHUNK, D), jnp.float32) for _ in range(NBUF)],
            pltpu.VMEM((NCHUNK, CHUNK), jnp.int32),
            pltpu.VMEM((ZROWS, D), jnp.float32),
            [pltpu.SemaphoreType.DMA for _ in range(NBUF)],
            [pltpu.SemaphoreType.DMA for _ in range(NBUF)],
            pltpu.SemaphoreType.DMA,
            pltpu.VMEM_SHARED((BATCH_PER_CORE * NUM_SEGMENTS, D), jnp.float32),
        ],
    )(v, seg32)
    return out
